# baseline (speedup 1.0000x reference)
import jax
import jax.numpy as jnp
import numpy as np
from jax.experimental import pallas as pl
from jax.experimental.pallas import tpu as pltpu

_B = 128
_CIN = 16
_T = 2048
_T4 = 512
_HID = 256
_OUT = 64
_KNN = 8
_F = 32
_NORM9 = float(np.float32(1.0 / 3.0) * np.float32(1.0 / 3.0))
_HI = jax.lax.Precision.HIGHEST


def _up(a):
    z1 = jnp.zeros((1, a.shape[1]), a.dtype)
    return jnp.concatenate([z1, a[:-1]], axis=0)


def _dn(a):
    z1 = jnp.zeros((1, a.shape[1]), a.dtype)
    return jnp.concatenate([a[1:], z1], axis=0)


def _dot(a, b):
    return jax.lax.dot(a, b, preferred_element_type=jnp.float32)


def _conv_group(x4, w1_ref, b1_ref, w2_ref, b2_ref):
    x0, x1, x2, x3 = (x4[:, j, :] for j in range(4))
    x0d, x1d, x2u, x3u = _dn(x0), _dn(x1), _up(x2), _up(x3)
    w = [w1_ref[k] for k in range(5)]
    c0 = (_dot(x2u, w[0]) + _dot(x3u, w[1]) + _dot(x0, w[2]) + _dot(x1, w[3])
          + _dot(x2, w[4]))
    c1 = (_dot(x3u, w[0]) + _dot(x0, w[1]) + _dot(x1, w[2]) + _dot(x2, w[3])
          + _dot(x3, w[4]))
    c2 = (_dot(x0, w[0]) + _dot(x1, w[1]) + _dot(x2, w[2]) + _dot(x3, w[3])
          + _dot(x0d, w[4]))
    c3 = (_dot(x1, w[0]) + _dot(x2, w[1]) + _dot(x3, w[2]) + _dot(x0d, w[3])
          + _dot(x1d, w[4]))
    he = jnp.maximum(jnp.maximum(c0, c1) + b1_ref[...], 0.0)
    ho = jnp.maximum(jnp.maximum(c2, c3) + b1_ref[...], 0.0)
    he = he.astype(jnp.bfloat16)
    ho = ho.astype(jnp.bfloat16)
    heu, hou, hed, hod = _up(he), _up(ho), _dn(he), _dn(ho)
    v = [w2_ref[k] for k in range(5)]
    y2e = (_dot(heu, v[0]) + _dot(hou, v[1]) + _dot(he, v[2]) + _dot(ho, v[3])
           + _dot(hed, v[4]))
    y2o = (_dot(hou, v[0]) + _dot(he, v[1]) + _dot(ho, v[2]) + _dot(hed, v[3])
           + _dot(hod, v[4]))
    return jnp.maximum(jnp.maximum(y2e, y2o) + b2_ref[...], 0.0)


def _gcn_batches(h2b, w1_ref, b1_ref, w2_ref, b2_ref, fcw_ref, fcb_ref,
                 m0=None):
    pooled = []
    for bsub in range(8):
        xb = h2b[:, bsub * _F:(bsub + 1) * _F]
        xw = _dot(xb, w1_ref[...])
        if m0 is not None and bsub == 0:
            xw = jax.lax.dot(m0, xw, precision=_HI,
                             preferred_element_type=jnp.float32)
        h1 = jnp.maximum(xw + b1_ref[...], 0.0).astype(jnp.bfloat16)
        xw2 = _dot(h1, w2_ref[...])
        if m0 is not None and bsub == 0:
            xw2 = jax.lax.dot(m0, xw2, precision=_HI,
                              preferred_element_type=jnp.float32)
        hg = jnp.maximum(xw2 + b2_ref[...], 0.0)
        pooled.append(jnp.sum(hg, axis=0, keepdims=True) * (1.0 / _T4))
    pooled = jnp.concatenate(pooled, axis=0)
    return _dot(pooled.astype(jnp.bfloat16), fcw_ref[...]) + fcb_ref[...]


def _mega_body(x4_ref, w1_ref, b1_ref, w2_ref, b2_ref, W1_ref, b1r_ref,
               W2_ref, b2r_ref, fcw_ref, fcb_ref, out_ref,
               mean_ref, comp_ref, h0_ref):
    g = pl.program_id(0)
    last = pl.num_programs(0) - 1

    @pl.when(g <= last - 1)
    def _():
        h2 = _conv_group(x4_ref[...], w1_ref, b1_ref, w2_ref, b2_ref)
        sel = (jax.lax.broadcasted_iota(jnp.int32, (256, _F), 0) % _F
               == jax.lax.broadcasted_iota(jnp.int32, (256, _F), 1))
        part = jax.lax.dot(h2, sel.astype(jnp.float32), precision=_HI,
                           preferred_element_type=jnp.float32) * (1.0 / _B)

        @pl.when(g == 0)
        def _():
            mean_ref[...] = part
            comp_ref[...] = jnp.zeros((_T4, _F), jnp.float32)

        @pl.when(g > 0)
        def _():
            y = part - comp_ref[...]
            s = mean_ref[...]
            t = s + y
            comp_ref[...] = (t - s) - y
            mean_ref[...] = t

        @pl.when(g == last - 1)
        def _():
            h0_ref[...] = h2

        @pl.when(g < last - 1)
        def _():
            out_ref[...] = _gcn_batches(h2.astype(jnp.bfloat16), W1_ref,
                                        b1r_ref, W2_ref, b2r_ref, fcw_ref,
                                        fcb_ref)

    @pl.when(g == last)
    def _():
        mf = mean_ref[...]
        mm = mf * mf
        sq_col = jax.lax.dot(mm, jnp.ones((_F, 1), jnp.float32),
                             precision=_HI,
                             preferred_element_type=jnp.float32)
        sq_row = jax.lax.dot_general(jnp.ones((1, _F), jnp.float32), mm,
                                     (((1,), (1,)), ((), ())), precision=_HI,
                                     preferred_element_type=jnp.float32)
        mfb = mf.astype(jnp.bfloat16)
        gram = jax.lax.dot_general(mfb, mfb, (((1,), (1,)), ((), ())),
                                   preferred_element_type=jnp.float32)
        rows = jax.lax.broadcasted_iota(jnp.int32, (_T4, _T4), 0)
        cols = jax.lax.broadcasted_iota(jnp.int32, (_T4, _T4), 1)
        eye = rows == cols
        d = (sq_col + sq_row) - 2.0 * gram + jnp.where(eye, 1e9, 0.0)
        m0 = jnp.where(eye, 1.0, 0.0)
        for _i in range(_KNN):
            mn = jnp.min(d, axis=1, keepdims=True)
            idx = jnp.min(jnp.where(d == mn, cols, jnp.int32(1 << 30)),
                          axis=1, keepdims=True)
            hit = cols == idx
            m0 = m0 + jnp.where(hit, 1.0, 0.0)
            d = jnp.where(hit, jnp.float32(3e38), d)
        m0 = m0 * _NORM9
        out_ref[...] = _gcn_batches(h0_ref[...].astype(jnp.bfloat16), W1_ref,
                                    b1r_ref, W2_ref, b2r_ref, fcw_ref,
                                    fcb_ref, m0=m0)


def kernel(x, conv1_w, conv1_b, conv2_w, conv2_b, W1, b1, W2, b2, fc_w, fc_b):
    f32 = jnp.float32
    bf16 = jnp.bfloat16
    xt = jnp.transpose(x, (2, 0, 1)).reshape(_T, _B * _CIN)
    x4 = xt.astype(bf16).reshape(_T4, 4, _B * _CIN)
    eye8 = jnp.eye(8, dtype=f32)
    w1s = jnp.stack([jnp.kron(eye8, conv1_w[:, :, k].T) for k in range(5)])
    w2s = jnp.stack([jnp.kron(eye8, conv2_w[:, :, k].T) for k in range(5)])
    w1s = w1s.astype(bf16)
    w2s = w2s.astype(bf16)
    b1t = jnp.tile(conv1_b, 8).reshape(1, 128)
    b2t = jnp.tile(conv2_b, 8).reshape(1, 256)

    n_groups = _B * _CIN // 128
    full = lambda g: (0, 0)
    out = pl.pallas_call(
        _mega_body,
        grid=(n_groups + 1,),
        in_specs=[
            pl.BlockSpec((_T4, 4, 128),
                         lambda g: (0, 0, jnp.where(g < 15, g + 1, 0))),
            pl.BlockSpec((5, 128, 128), lambda g: (0, 0, 0)),
            pl.BlockSpec((1, 128), full),
            pl.BlockSpec((5, 128, 256), lambda g: (0, 0, 0)),
            pl.BlockSpec((1, 256), full),
            pl.BlockSpec((_F, _HID), full),
            pl.BlockSpec((1, _HID), full),
            pl.BlockSpec((_HID, _HID), full),
            pl.BlockSpec((1, _HID), full),
            pl.BlockSpec((_HID, _OUT), full),
            pl.BlockSpec((1, _OUT), full),
        ],
        out_specs=pl.BlockSpec((8, _OUT),
                               lambda g: (jnp.where(g < 15, g + 1, 0), 0)),
        out_shape=jax.ShapeDtypeStruct((_B, _OUT), f32),
        scratch_shapes=[
            pltpu.VMEM((_T4, _F), f32),
            pltpu.VMEM((_T4, _F), f32),
            pltpu.VMEM((_T4, 256), f32),
        ],
        compiler_params=pltpu.CompilerParams(
            dimension_semantics=("arbitrary",)),
    )(x4, w1s, b1t, w2s, b2t, W1.astype(bf16), b1.reshape(1, _HID),
      W2.astype(bf16), b2.reshape(1, _HID), fc_w.astype(bf16),
      fc_b.reshape(1, _OUT))
    return out


# speedup vs baseline: 2.0363x; 2.0363x over previous
import jax
import jax.numpy as jnp
import numpy as np
from jax.experimental import pallas as pl
from jax.experimental.pallas import tpu as pltpu

_B = 128
_CIN = 16
_T = 2048
_T4 = 512
_HID = 256
_OUT = 64
_KNN = 8
_F = 32
_NORM9 = float(np.float32(1.0 / 3.0) * np.float32(1.0 / 3.0))
_HI = jax.lax.Precision.HIGHEST


def _up(a):
    z1 = jnp.zeros((1, a.shape[1]), a.dtype)
    return jnp.concatenate([z1, a[:-1]], axis=0)


def _dn(a):
    z1 = jnp.zeros((1, a.shape[1]), a.dtype)
    return jnp.concatenate([a[1:], z1], axis=0)


def _dot(a, b):
    return jax.lax.dot(a, b, preferred_element_type=jnp.float32)


def _conv_group(xb, w1_ref, b1_ref, w2_ref, b2_ref):
    xbb = xb.astype(jnp.bfloat16)
    ident = (jax.lax.broadcasted_iota(jnp.int32, (128, 128), 0)
             == jax.lax.broadcasted_iota(jnp.int32, (128, 128), 1))
    xt = jax.lax.dot_general(xbb, ident.astype(jnp.bfloat16),
                             (((0,), (0,)), ((), ())),
                             preferred_element_type=jnp.float32)
    x4 = xt.astype(jnp.bfloat16).reshape(_T4, 4, 128)
    x0, x1, x2, x3 = (x4[:, j, :] for j in range(4))
    x0d, x1d, x2u, x3u = _dn(x0), _dn(x1), _up(x2), _up(x3)
    w = [w1_ref[k] for k in range(5)]
    c0 = (_dot(x2u, w[0]) + _dot(x3u, w[1]) + _dot(x0, w[2]) + _dot(x1, w[3])
          + _dot(x2, w[4]))
    c1 = (_dot(x3u, w[0]) + _dot(x0, w[1]) + _dot(x1, w[2]) + _dot(x2, w[3])
          + _dot(x3, w[4]))
    c2 = (_dot(x0, w[0]) + _dot(x1, w[1]) + _dot(x2, w[2]) + _dot(x3, w[3])
          + _dot(x0d, w[4]))
    c3 = (_dot(x1, w[0]) + _dot(x2, w[1]) + _dot(x3, w[2]) + _dot(x0d, w[3])
          + _dot(x1d, w[4]))
    he = jnp.maximum(jnp.maximum(c0, c1) + b1_ref[...], 0.0)
    ho = jnp.maximum(jnp.maximum(c2, c3) + b1_ref[...], 0.0)
    he = he.astype(jnp.bfloat16)
    ho = ho.astype(jnp.bfloat16)
    heu, hou, hed, hod = _up(he), _up(ho), _dn(he), _dn(ho)
    v = [w2_ref[k] for k in range(5)]
    y2e = (_dot(heu, v[0]) + _dot(hou, v[1]) + _dot(he, v[2]) + _dot(ho, v[3])
           + _dot(hed, v[4]))
    y2o = (_dot(hou, v[0]) + _dot(he, v[1]) + _dot(ho, v[2]) + _dot(hed, v[3])
           + _dot(hod, v[4]))
    return jnp.maximum(jnp.maximum(y2e, y2o) + b2_ref[...], 0.0)


def _gcn_batches(h2b, w1_ref, b1_ref, w2_ref, b2_ref, fcw_ref, fcb_ref,
                 m0=None):
    pooled = []
    for bsub in range(8):
        xb = h2b[:, bsub * _F:(bsub + 1) * _F]
        xw = _dot(xb, w1_ref[...])
        if m0 is not None and bsub == 0:
            xw = jax.lax.dot(m0, xw, precision=_HI,
                             preferred_element_type=jnp.float32)
        h1 = jnp.maximum(xw + b1_ref[...], 0.0).astype(jnp.bfloat16)
        xw2 = _dot(h1, w2_ref[...])
        if m0 is not None and bsub == 0:
            xw2 = jax.lax.dot(m0, xw2, precision=_HI,
                              preferred_element_type=jnp.float32)
        hg = jnp.maximum(xw2 + b2_ref[...], 0.0)
        pooled.append(jnp.sum(hg, axis=0, keepdims=True) * (1.0 / _T4))
    pooled = jnp.concatenate(pooled, axis=0)
    return _dot(pooled.astype(jnp.bfloat16), fcw_ref[...]) + fcb_ref[...]


def _mega_body(x2d_ref, w1_ref, b1_ref, w2_ref, b2_ref, W1_ref, b1r_ref,
               W2_ref, b2r_ref, fcw_ref, fcb_ref, out_ref,
               mean_ref, comp_ref, h0_ref):
    g = pl.program_id(0)
    last = pl.num_programs(0) - 1

    @pl.when(g <= last - 1)
    def _():
        h2 = _conv_group(x2d_ref[...], w1_ref, b1_ref, w2_ref, b2_ref)
        part = jnp.sum(h2.reshape(_T4, 8, _F), axis=1) * (1.0 / _B)

        @pl.when(g == 0)
        def _():
            mean_ref[...] = part
            comp_ref[...] = jnp.zeros((_T4, _F), jnp.float32)

        @pl.when(g > 0)
        def _():
            y = part - comp_ref[...]
            s = mean_ref[...]
            t = s + y
            comp_ref[...] = (t - s) - y
            mean_ref[...] = t

        @pl.when(g == last - 1)
        def _():
            h0_ref[...] = h2

        @pl.when(g < last - 1)
        def _():
            out_ref[...] = _gcn_batches(h2.astype(jnp.bfloat16), W1_ref,
                                        b1r_ref, W2_ref, b2r_ref, fcw_ref,
                                        fcb_ref)

    @pl.when(g == last)
    def _():
        mf = mean_ref[...]
        mm = mf * mf
        sq_col = jax.lax.dot(mm, jnp.ones((_F, 1), jnp.float32),
                             precision=_HI,
                             preferred_element_type=jnp.float32)
        sq_row = jax.lax.dot_general(jnp.ones((1, _F), jnp.float32), mm,
                                     (((1,), (1,)), ((), ())), precision=_HI,
                                     preferred_element_type=jnp.float32)
        mfb = mf.astype(jnp.bfloat16)
        gram = jax.lax.dot_general(mfb, mfb, (((1,), (1,)), ((), ())),
                                   preferred_element_type=jnp.float32)
        rows = jax.lax.broadcasted_iota(jnp.int32, (_T4, _T4), 0)
        cols = jax.lax.broadcasted_iota(jnp.int32, (_T4, _T4), 1)
        eye = rows == cols
        d = (sq_col + sq_row) - 2.0 * gram + jnp.where(eye, 1e9, 0.0)
        m0 = jnp.where(eye, 1.0, 0.0)
        for _i in range(_KNN):
            mn = jnp.min(d, axis=1, keepdims=True)
            idx = jnp.min(jnp.where(d == mn, cols, jnp.int32(1 << 30)),
                          axis=1, keepdims=True)
            hit = cols == idx
            m0 = m0 + jnp.where(hit, 1.0, 0.0)
            d = jnp.where(hit, jnp.float32(3e38), d)
        m0 = m0 * _NORM9
        out_ref[...] = _gcn_batches(h0_ref[...].astype(jnp.bfloat16), W1_ref,
                                    b1r_ref, W2_ref, b2r_ref, fcw_ref,
                                    fcb_ref, m0=m0)


def kernel(x, conv1_w, conv1_b, conv2_w, conv2_b, W1, b1, W2, b2, fc_w, fc_b):
    f32 = jnp.float32
    bf16 = jnp.bfloat16
    x2d = x.reshape(_B * _CIN, _T)
    eye8 = jnp.eye(8, dtype=f32)
    w1s = jnp.stack([jnp.kron(eye8, conv1_w[:, :, k].T) for k in range(5)])
    w2s = jnp.stack([jnp.kron(eye8, conv2_w[:, :, k].T) for k in range(5)])
    w1s = w1s.astype(bf16)
    w2s = w2s.astype(bf16)
    b1t = jnp.tile(conv1_b, 8).reshape(1, 128)
    b2t = jnp.tile(conv2_b, 8).reshape(1, 256)

    n_groups = _B * _CIN // 128
    full = lambda g: (0, 0)
    out = pl.pallas_call(
        _mega_body,
        grid=(n_groups + 1,),
        in_specs=[
            pl.BlockSpec((128, _T),
                         lambda g: (jnp.where(g < 15, g + 1, 0), 0)),
            pl.BlockSpec((5, 128, 128), lambda g: (0, 0, 0)),
            pl.BlockSpec((1, 128), full),
            pl.BlockSpec((5, 128, 256), lambda g: (0, 0, 0)),
            pl.BlockSpec((1, 256), full),
            pl.BlockSpec((_F, _HID), full),
            pl.BlockSpec((1, _HID), full),
            pl.BlockSpec((_HID, _HID), full),
            pl.BlockSpec((1, _HID), full),
            pl.BlockSpec((_HID, _OUT), full),
            pl.BlockSpec((1, _OUT), full),
        ],
        out_specs=pl.BlockSpec((8, _OUT),
                               lambda g: (jnp.where(g < 15, g + 1, 0), 0)),
        out_shape=jax.ShapeDtypeStruct((_B, _OUT), f32),
        scratch_shapes=[
            pltpu.VMEM((_T4, _F), f32),
            pltpu.VMEM((_T4, _F), f32),
            pltpu.VMEM((_T4, 256), f32),
        ],
        compiler_params=pltpu.CompilerParams(
            dimension_semantics=("arbitrary",)),
    )(x2d, w1s, b1t, w2s, b2t, W1.astype(bf16), b1.reshape(1, _HID),
      W2.astype(bf16), b2.reshape(1, _HID), fc_w.astype(bf16),
      fc_b.reshape(1, _OUT))
    return out
